# baseline (speedup 1.0000x reference)
import functools

import jax
import jax.numpy as jnp
from jax import lax
from jax.experimental import pallas as pl
from jax.experimental.pallas import tpu as pltpu
from jax.experimental.pallas import tpu_sc as plsc

_CONF = 0.01
_NMS_THR = 0.5
_TOP_K = 10
_LEN_THR = 3.0
_LO, _HI = 0.0, 416.0

_B = 16
_N = 20000
_CH = 4000
_NCH = _N // _CH
_CAP = _N + 32
_QUADS = _CH // 64
_REM_GROUPS = (_CH - _QUADS * 64) // 16


def _vf(x):
    return jnp.full((16,), x, jnp.float32)


def _vi(x):
    return jnp.full((16,), x, jnp.int32)


def _nms_body(s_hbm, reg_hbm, box_hbm, out_hbm,
              bs0, brg0, bbx0, bs1, brg1, bbx1,
              cs, c1, c2, det, sem0, sem1):
    sid = lax.axis_index("s")
    b = sid

    bufs = ((bs0, brg0, bbx0), (bs1, brg1, bbx1))
    sems = (sem0, sem1)

    lanes = lax.iota(jnp.int32, 16)
    lanes2 = lanes * _vi(2)
    lanes2p1 = lanes2 + _vi(1)
    neg1 = _vf(-1.0)
    det[pl.ds(0, 16)] = neg1
    det[pl.ds(16, 16)] = neg1

    def issue(ch, slot):
        off = b * _N + ch * _CH
        return [
            pltpu.async_copy(s_hbm.at[pl.ds(off, _CH)], bufs[slot][0],
                             sems[slot]),
            pltpu.async_copy(reg_hbm.at[pl.ds(2 * off, 2 * _CH)],
                             bufs[slot][1], sems[slot]),
            pltpu.async_copy(box_hbm.at[pl.ds(2 * off, 2 * _CH)],
                             bufs[slot][2], sems[slot]),
        ]

    def groups_body(bufset, base, cntm1, n):
        bsb, brgb, bbxb = bufset
        svs = [bsb[pl.ds(base + u * 16, 16)] for u in range(n)]
        idx0s = [_vi(2 * base + 32 * u) + lanes2 for u in range(n)]
        idx1s = [_vi(2 * base + 32 * u) + lanes2p1 for u in range(n)]
        dxs = [plsc.load_gather(brgb, [i0]) for i0 in idx0s]
        dws = [plsc.load_gather(brgb, [i1]) for i1 in idx1s]
        p1s = [plsc.load_gather(bbxb, [i0]) for i0 in idx0s]
        p2s = [plsc.load_gather(bbxb, [i1]) for i1 in idx1s]
        es = [jnp.exp(dw) for dw in dws]
        ws = [p2 - p1 for p1, p2 in zip(p1s, p2s)]
        ctrs = [p1 + _vf(0.5) * w for p1, w in zip(p1s, ws)]
        pcs = [ctr + dx * w for ctr, dx, w in zip(ctrs, dxs, ws)]
        hws = [_vf(0.5) * (e * w) for e, w in zip(es, ws)]
        x1s = [jnp.minimum(jnp.maximum(pc - hw, _vf(_LO)), _vf(_HI))
               for pc, hw in zip(pcs, hws)]
        x2s = [jnp.minimum(jnp.maximum(pc + hw, _vf(_LO)), _vf(_HI))
               for pc, hw in zip(pcs, hws)]
        ms = [(sv > _vf(_CONF)) & ((x2 - x1) > _vf(_LEN_THR))
              for sv, x1, x2 in zip(svs, x1s, x2s)]
        csums = [plsc.cumsum(m.astype(jnp.int32)) for m in ms]
        pops = [plsc.all_reduce_population_count(m) for m in ms]
        cnts = [cntm1]
        for u in range(n - 1):
            cnts.append(cnts[u] + pops[u])
        idxs = [cnts[u] + csums[u] for u in range(n)]
        for u in range(n):
            plsc.store_scatter(cs, [idxs[u]], svs[u], mask=ms[u])
            plsc.store_scatter(c1, [idxs[u]], x1s[u], mask=ms[u])
            plsc.store_scatter(c2, [idxs[u]], x2s[u], mask=ms[u])
        return cnts[n - 1] + pops[n - 1]

    cntm1 = _vi(-1)
    descs = issue(0, 0)
    for ch in range(_NCH):
        slot = ch % 2
        nxt = issue(ch + 1, 1 - slot) if ch + 1 < _NCH else None
        for d in descs:
            d.wait()
        bufset = bufs[slot]

        def quad(q, cntm1, bufset=bufset):
            return groups_body(bufset, q * 64, cntm1, 4)

        cntm1 = lax.fori_loop(0, _QUADS, quad, cntm1)
        for u in range(_REM_GROUPS):
            cntm1 = groups_body(bufset, _QUADS * 64 + u * 16, cntm1, 1)
        descs = nxt

    cnt = jnp.max(cntm1) + 1
    plsc.store_scatter(cs, [_vi(cnt) + lanes], neg1)
    nsteps = (cnt + 15) // 16

    def pick(t, carry):
        kprev, x1p, x2p = carry
        kpv = _vi(kprev)
        x1pv = _vf(x1p)
        x2pv = _vf(x2p)
        lpv = x2pv - x1pv

        def fused(j, st):
            bv, bi = st
            sl = pl.ds(j * 16, 16)
            sv = cs[sl]
            a1 = c1[sl]
            a2 = c2[sl]
            inter = jnp.maximum(
                jnp.minimum(x2pv, a2) - jnp.maximum(x1pv, a1), _vf(0.0))
            union = lpv + (a2 - a1) - inter
            iou = inter / jnp.maximum(union, _vf(1e-12))
            iv = _vi(j * 16) + lanes
            kill = (iou > _vf(_NMS_THR)) | (iv == kpv)
            sv = jnp.where(kill, neg1, sv)
            cs[sl] = sv
            upd = sv > bv
            return jnp.where(upd, sv, bv), jnp.where(upd, iv, bi)

        bv, bi = lax.fori_loop(0, nsteps, fused, (neg1, _vi(0)))
        mx = jnp.max(bv)
        cand = jnp.where(bv == _vf(mx), bi, _vi(2 ** 30))
        k = jnp.min(cand)

        def emit():
            kv = _vi(k)
            x1kv = plsc.load_gather(c1, [kv])
            x2kv = plsc.load_gather(c2, [kv])
            skv = plsc.load_gather(cs, [kv])
            val = jnp.where(lanes == _vi(0), x1kv,
                            jnp.where(lanes == _vi(1), x2kv, skv))
            plsc.store_scatter(det, [_vi(3 * t) + lanes], val,
                               mask=lanes < _vi(3))
            return k, jnp.max(x1kv), jnp.max(x2kv)

        def skip():
            return jnp.int32(-1), jnp.float32(0.0), jnp.float32(0.0)

        return lax.cond(mx > 0.0, emit, skip)

    lax.fori_loop(0, _TOP_K, pick,
                  (jnp.int32(-1), jnp.float32(0.0), jnp.float32(0.0)))
    pltpu.sync_copy(det, out_hbm.at[pl.ds(b * 32, 32)])


_sc_nms = functools.partial(
    pl.kernel,
    out_type=jax.ShapeDtypeStruct((_B * 32,), jnp.float32),
    mesh=plsc.VectorSubcoreMesh(core_axis_name="c", subcore_axis_name="s",
                                num_cores=1, num_subcores=16),
    scratch_types=[
        pltpu.VMEM((_CH,), jnp.float32),
        pltpu.VMEM((2 * _CH,), jnp.float32),
        pltpu.VMEM((2 * _CH,), jnp.float32),
        pltpu.VMEM((_CH,), jnp.float32),
        pltpu.VMEM((2 * _CH,), jnp.float32),
        pltpu.VMEM((2 * _CH,), jnp.float32),
        pltpu.VMEM((_CAP,), jnp.float32),
        pltpu.VMEM((_CAP,), jnp.float32),
        pltpu.VMEM((_CAP,), jnp.float32),
        pltpu.VMEM((32,), jnp.float32),
        pltpu.SemaphoreType.DMA,
        pltpu.SemaphoreType.DMA,
    ],
    compiler_params=pltpu.CompilerParams(needs_layout_passes=False),
)(_nms_body)


@jax.jit
def kernel(clf_proba, reg_preds_all, all_proposal_boxes, device):
    del device
    scores = clf_proba.reshape(-1)
    reg = reg_preds_all.reshape(-1)
    box = all_proposal_boxes.reshape(-1)
    out = _sc_nms(scores, reg, box)
    return out.reshape(_B, 32)[:, :_TOP_K * 3].reshape(_B, _TOP_K, 3)
